# baseline (speedup 1.0000x reference)
import jax
import jax.numpy as jnp
from jax import lax
from jax.experimental import pallas as pl
from jax.experimental.pallas import tpu as pltpu
from jax.experimental.pallas import tpu_sc as plsc

N_Q = 32
DIM = 128
M_KV = 32768
K_TOPK = 256


_BM = 2048


def _tc_scores_body(q_ref, k_ref, o_ref):
    qp = q_ref[...]
    kb = k_ref[...]
    s = lax.dot_general(qp, kb, (((1,), (1,)), ((), ())),
                        preferred_element_type=jnp.float32)
    o_ref[...] = jnp.max(s, axis=0)


def _tc_scores(qp, keys):
    return pl.pallas_call(
        _tc_scores_body,
        grid=(M_KV // _BM,),
        in_specs=[
            pl.BlockSpec((N_Q, DIM), lambda i: (0, 0)),
            pl.BlockSpec((_BM, DIM), lambda i: (i, 0)),
        ],
        out_specs=pl.BlockSpec((_BM,), lambda i: (i,)),
        out_shape=jax.ShapeDtypeStruct((M_KV,), jnp.float32),
    )(qp, keys)


_NT = 16
_CHUNK = M_KV // _NT
_NV = _CHUNK // 16
_MSB = -2**31
_GT_CAP = 288
_EQ_CAP = 2064

_O_HIST = 0
_O_CGT = 256
_O_CEQ = 512
_O_GTK = 768
_O_GTI = 4864
_O_EQI = 8960
_O_CNK = 13056
_O_CNI = 13568
_O_RNK = 14080
_O_RIX = 14336
_O_WSM = 14592
_O_ENC = 14848
_SH_WORDS = 16896


def _sc_body(scores_hbm, values_hbm, out_enc, out_idx,
             sc_scores, sc_keys, gt_idx, gt_key, eq_idx,
             cand_key, cand_idx, histbuf, hist2d,
             rank_ref, widx_ref, gidx_ref, wbuf, vrows, acc, acc_i,
             fillbuf, fillb2, gtk_all, gti_all, eqi_all,
             sh, dma_sem):
    cid = lax.axis_index("c")
    sid = lax.axis_index("s")
    tid = sid * 1 + cid
    base = tid * _CHUNK
    iota = lax.iota(jnp.int32, 16)

    pltpu.sync_copy(scores_hbm.at[pl.ds(base, _CHUNK)], sc_scores)

    def _mk_keys(i, carry):
        s = sc_scores[pl.ds(i * 16, 16)]
        b = lax.bitcast_convert_type(s, jnp.int32)
        pat = jnp.where(s >= 0.0, b | _MSB, ~b)
        sc_keys[pl.ds(i * 16, 16)] = pat
        return carry

    lax.fori_loop(0, _NV, _mk_keys, jnp.int32(0))

    need = jnp.int32(K_TOPK)
    prefix = jnp.int32(0)
    ones16 = jnp.ones((16,), jnp.int32)
    for r in range(8):
        shift = 28 - 4 * r
        plsc.subcore_barrier()
        for j in range(16):
            hist2d[j] = jnp.zeros((16,), jnp.int32)

        if r == 0:
            def _hist0(i, carry):
                k = sc_keys[pl.ds(i * 16, 16)]
                digit = lax.shift_right_logical(k, 28) & 0xF
                plsc.addupdate_scatter(hist2d, [digit, iota], ones16)
                return carry
            lax.fori_loop(0, _NV, _hist0, jnp.int32(0))
        else:
            def _hist(i, carry):
                pfx = carry
                k = sc_keys[pl.ds(i * 16, 16)]
                active = lax.shift_right_logical(k, shift + 4) == pfx
                digit = lax.shift_right_logical(k, shift) & 0xF
                plsc.addupdate_scatter(hist2d, [digit, iota], ones16,
                                       mask=active)
                return pfx
            lax.fori_loop(0, _NV, _hist, prefix)

        tl = jnp.zeros((16,), jnp.int32)
        for j in range(16):
            tl = tl + plsc.load_gather(hist2d, [iota, jnp.full((16,), j,
                                                               jnp.int32)])
        histbuf[...] = tl

        pltpu.sync_copy(histbuf, sh.at[pl.ds(_O_HIST + tid * 16, 16)])
        plsc.subcore_barrier()
        pltpu.sync_copy(sh.at[pl.ds(_O_HIST, 256)], fillbuf)
        total = jnp.zeros((16,), jnp.int32)
        for t in range(_NT):
            total = total + fillbuf[pl.ds(t * 16, 16)]
        cnt_ge = lax.rev(jnp.cumsum(lax.rev(total, (0,))), (0,))
        dsel = jnp.max(jnp.where(cnt_ge >= need, iota, -1))
        cnt_gt = jnp.sum(jnp.where(iota == dsel, cnt_ge - total, 0))
        need = need - cnt_gt
        prefix = (prefix << 4) | dsel

    T = prefix
    need_eq = need
    A = jnp.int32(K_TOPK) - need_eq

    Tx = T ^ _MSB

    def _select(i, carry):
        c_gt, c_eq = carry
        k = sc_keys[pl.ds(i * 16, 16)]
        kx = k ^ _MSB
        m_gt = kx > Tx
        m_eq = k == T
        gidx = base + i * 16 + iota
        mi_gt = m_gt.astype(jnp.int32)
        mi_eq = m_eq.astype(jnp.int32)
        pos_gt = c_gt + plsc.cumsum(mi_gt) - mi_gt
        pos_eq = c_eq + plsc.cumsum(mi_eq) - mi_eq
        m_gtw = m_gt & (pos_gt < 256)
        m_eqw = m_eq & (pos_eq < _EQ_CAP - 16)
        plsc.store_scatter(gt_idx, [pos_gt], gidx, mask=m_gtw)
        plsc.store_scatter(gt_key, [pos_gt], k, mask=m_gtw)
        plsc.store_scatter(eq_idx, [pos_eq], gidx, mask=m_eqw)
        c_gt = c_gt + jnp.sum(mi_gt)
        c_eq = c_eq + jnp.sum(mi_eq)
        return (c_gt, c_eq)

    c_gt, c_eq = lax.fori_loop(0, _NV, _select,
                               (jnp.int32(0), jnp.int32(0)))

    histbuf[...] = jnp.full((16,), c_gt, jnp.int32)
    pltpu.sync_copy(histbuf, sh.at[pl.ds(_O_CGT + tid * 16, 16)])
    histbuf[...] = jnp.full((16,), c_eq, jnp.int32)
    pltpu.sync_copy(histbuf, sh.at[pl.ds(_O_CEQ + tid * 16, 16)])
    pltpu.sync_copy(gt_key.at[pl.ds(0, 256)],
                    sh.at[pl.ds(_O_GTK + tid * 256, 256)])
    pltpu.sync_copy(gt_idx.at[pl.ds(0, 256)],
                    sh.at[pl.ds(_O_GTI + tid * 256, 256)])
    pltpu.sync_copy(eq_idx.at[pl.ds(0, 256)],
                    sh.at[pl.ds(_O_EQI + tid * 256, 256)])
    plsc.subcore_barrier()

    @pl.when(tid == 0)
    def _compact():
        pltpu.sync_copy(sh.at[pl.ds(_O_CGT, 256)], fillbuf)
        pltpu.sync_copy(sh.at[pl.ds(_O_CEQ, 256)], fillb2)
        pltpu.sync_copy(sh.at[pl.ds(_O_GTK, 4096)], gtk_all)
        pltpu.sync_copy(sh.at[pl.ds(_O_GTI, 4096)], gti_all)
        pltpu.sync_copy(sh.at[pl.ds(_O_EQI, 4096)], eqi_all)
        for i in range(16):
            cand_key[pl.ds(i * 16, 16)] = jnp.full((16,), T, jnp.int32)
        off = jnp.int32(0)
        for t in range(_NT):
            cg = jnp.max(fillbuf[pl.ds(t * 16, 16)])
            for j in range(16):
                p = j * 16 + iota
                pos = off + p
                m = (p < cg) & (pos >= 0) & (pos < 256)
                plsc.store_scatter(cand_key, [pos],
                                   gtk_all[pl.ds(t * 256 + j * 16, 16)],
                                   mask=m)
                plsc.store_scatter(cand_idx, [pos],
                                   gti_all[pl.ds(t * 256 + j * 16, 16)],
                                   mask=m)
            off = off + cg
        oeq = jnp.int32(0)
        for t in range(_NT):
            ce = jnp.max(fillb2[pl.ds(t * 16, 16)])
            for j in range(16):
                p = j * 16 + iota
                gpos = oeq + p
                pos = A + gpos
                m = ((p < ce) & (gpos < need_eq)
                     & (pos >= 0) & (pos < 256))
                plsc.store_scatter(cand_idx, [pos],
                                   eqi_all[pl.ds(t * 256 + j * 16, 16)],
                                   mask=m)
            oeq = oeq + ce
        for i in range(16):
            cand_key[pl.ds(256 + i * 16, 16)] = cand_key[pl.ds(i * 16, 16)]
            cand_idx[pl.ds(256 + i * 16, 16)] = cand_idx[pl.ds(i * 16, 16)]
        pltpu.sync_copy(cand_key, sh.at[pl.ds(_O_CNK, 512)])
        pltpu.sync_copy(cand_idx, sh.at[pl.ds(_O_CNI, 512)])

    plsc.subcore_barrier()
    pltpu.sync_copy(sh.at[pl.ds(_O_CNK, 512)], cand_key)
    pltpu.sync_copy(sh.at[pl.ds(_O_CNI, 512)], cand_idx)

    mybase = tid * 16
    key_me = cand_key[pl.ds(mybase, 16)]
    idx_me = cand_idx[pl.ds(mybase, 16)]
    kx_me = key_me ^ _MSB

    def _rank(s, rk):
        ks = cand_key[pl.ds(mybase + s, 16)]
        is_ = cand_idx[pl.ds(mybase + s, 16)]
        gt = (ks ^ _MSB) > kx_me
        tie = (ks == key_me) & (is_ < idx_me)
        return rk + (gt | tie).astype(jnp.int32)

    rank_me = lax.fori_loop(0, 256, _rank, jnp.zeros((16,), jnp.int32))

    smax = jnp.float32(-jnp.inf)
    for i in range(16):
        kk = cand_key[pl.ds(i * 16, 16)]
        b = jnp.where(kk < 0, kk ^ _MSB, ~kk)
        sv = lax.bitcast_convert_type(b, jnp.float32)
        smax = jnp.maximum(smax, jnp.max(sv))
    b_me = jnp.where(key_me < 0, key_me ^ _MSB, ~key_me)
    s_me = lax.bitcast_convert_type(b_me, jnp.float32)
    w_me = jnp.exp(s_me - smax)

    rank_ref[...] = rank_me
    widx_ref[...] = idx_me
    wbuf[...] = w_me
    pltpu.sync_copy(rank_ref, sh.at[pl.ds(_O_RNK + tid * 16, 16)])
    pltpu.sync_copy(widx_ref, sh.at[pl.ds(_O_RIX + tid * 16, 16)])
    histbuf[...] = lax.bitcast_convert_type(
        jnp.full((16,), jnp.sum(w_me), jnp.float32), jnp.int32)
    pltpu.sync_copy(histbuf, sh.at[pl.ds(_O_WSM + tid * 16, 16)])

    gidx_ref[...] = jnp.minimum(jnp.maximum(idx_me, 0), M_KV - 1)
    pltpu.async_copy(values_hbm.at[gidx_ref], vrows, dma_sem).wait()
    wv = wbuf[...]
    for c in range(8):
        acc[pl.ds(c * 16, 16)] = jnp.zeros((16,), jnp.float32)
    for l in range(16):
        wl = wv[l]
        for c in range(8):
            acc[pl.ds(c * 16, 16)] = (acc[pl.ds(c * 16, 16)]
                                      + wl * vrows[l, pl.ds(c * 16, 16)])
    for c in range(8):
        acc_i[pl.ds(c * 16, 16)] = lax.bitcast_convert_type(
            acc[pl.ds(c * 16, 16)], jnp.int32)
    pltpu.sync_copy(acc_i, sh.at[pl.ds(_O_ENC + tid * 128, 128)])
    plsc.subcore_barrier()

    @pl.when(tid == 0)
    def _finish():
        pltpu.sync_copy(sh.at[pl.ds(_O_WSM, 256)], fillbuf)
        denom = jnp.float32(0.0)
        for t in range(_NT):
            row = lax.bitcast_convert_type(fillbuf[pl.ds(t * 16, 16)],
                                           jnp.float32)
            denom = denom + jnp.max(row)
        pltpu.sync_copy(sh.at[pl.ds(_O_ENC, 2048)], gti_all.at[pl.ds(0, 2048)])
        for c in range(8):
            tot = jnp.zeros((16,), jnp.float32)
            for t in range(_NT):
                tot = tot + lax.bitcast_convert_type(
                    gti_all[pl.ds(t * 128 + c * 16, 16)], jnp.float32)
            acc[pl.ds(c * 16, 16)] = tot / denom
        pltpu.sync_copy(acc, out_enc)
        pltpu.sync_copy(sh.at[pl.ds(_O_RNK, 256)], fillbuf)
        pltpu.sync_copy(sh.at[pl.ds(_O_RIX, 256)], fillb2)
        for t in range(_NT):
            rv = fillbuf[pl.ds(t * 16, 16)]
            plsc.store_scatter(sc_keys, [rv], fillb2[pl.ds(t * 16, 16)],
                               mask=(rv >= 0) & (rv < 256))
        pltpu.sync_copy(sc_keys.at[pl.ds(0, 256)], out_idx)


def _sc_topk(scores, values):
    mesh = plsc.VectorSubcoreMesh(core_axis_name="c", subcore_axis_name="s",
                                  num_cores=1)
    f = pl.kernel(
        _sc_body,
        mesh=mesh,
        compiler_params=pltpu.CompilerParams(needs_layout_passes=False),
        out_type=[
            jax.ShapeDtypeStruct((DIM,), jnp.float32),
            jax.ShapeDtypeStruct((K_TOPK,), jnp.int32),
        ],
        scratch_types=[
            pltpu.VMEM((_CHUNK,), jnp.float32),
            pltpu.VMEM((_CHUNK,), jnp.int32),
            pltpu.VMEM((_GT_CAP,), jnp.int32),
            pltpu.VMEM((_GT_CAP,), jnp.int32),
            pltpu.VMEM((_EQ_CAP,), jnp.int32),
            pltpu.VMEM((512,), jnp.int32),
            pltpu.VMEM((512,), jnp.int32),
            pltpu.VMEM((16,), jnp.int32),
            pltpu.VMEM((16, 16), jnp.int32),
            pltpu.VMEM((16,), jnp.int32),
            pltpu.VMEM((16,), jnp.int32),
            pltpu.VMEM((16,), jnp.int32),
            pltpu.VMEM((16,), jnp.float32),
            pltpu.VMEM((16, DIM), jnp.float32),
            pltpu.VMEM((DIM,), jnp.float32),
            pltpu.VMEM((DIM,), jnp.int32),
            pltpu.VMEM((256,), jnp.int32),
            pltpu.VMEM((256,), jnp.int32),
            pltpu.VMEM((4096,), jnp.int32),
            pltpu.VMEM((4096,), jnp.int32),
            pltpu.VMEM((4096,), jnp.int32),
            pltpu.VMEM_SHARED((_SH_WORDS,), jnp.int32),
            pltpu.SemaphoreType.DMA,
        ],
    )
    return f(scores, values)


def kernel(queries, values, keys, affine):
    qp = jnp.matmul(queries, jnp.diag(affine))
    scores = _tc_scores(qp, keys)
    enc, idx = _sc_topk(scores, values)
    return enc, idx


# speedup vs baseline: 1.0000x; 1.0000x over previous
import jax
import jax.numpy as jnp
from jax import lax
from jax.experimental import pallas as pl
from jax.experimental.pallas import tpu as pltpu
from jax.experimental.pallas import tpu_sc as plsc

N_Q = 32
DIM = 128
M_KV = 32768
K_TOPK = 256


_BM = 2048


def _tc_scores_body(q_ref, k_ref, o_ref):
    qp = q_ref[...]
    kb = k_ref[...]
    s = lax.dot_general(qp, kb, (((1,), (1,)), ((), ())),
                        preferred_element_type=jnp.float32)
    o_ref[...] = jnp.max(s, axis=0)


def _tc_scores(qp, keys):
    return pl.pallas_call(
        _tc_scores_body,
        grid=(M_KV // _BM,),
        in_specs=[
            pl.BlockSpec((N_Q, DIM), lambda i: (0, 0)),
            pl.BlockSpec((_BM, DIM), lambda i: (i, 0)),
        ],
        out_specs=pl.BlockSpec((_BM,), lambda i: (i,)),
        out_shape=jax.ShapeDtypeStruct((M_KV,), jnp.float32),
    )(qp, keys)


_NT = 16
_CHUNK = M_KV // _NT
_NV = _CHUNK // 16
_MSB = -2**31
_GT_CAP = 288
_EQ_CAP = 2064

_O_HIST = 0
_O_CGT = 256
_O_CEQ = 512
_O_GTK = 768
_O_GTI = 4864
_O_EQI = 8960
_O_CNK = 13056
_O_CNI = 13568
_O_RNK = 14080
_O_RIX = 14336
_O_WSM = 14592
_O_ENC = 14848
_SH_WORDS = 16896


def _sc_body(scores_hbm, values_hbm, out_enc, out_idx,
             sc_scores, sc_keys, gt_idx, gt_key, eq_idx,
             cand_key, cand_idx, histbuf, hist2d,
             rank_ref, widx_ref, gidx_ref, wbuf, vrows, acc, acc_i,
             fillbuf, fillb2, gtk_all, gti_all, eqi_all,
             sh, dma_sem):
    cid = lax.axis_index("c")
    sid = lax.axis_index("s")
    tid = sid * 1 + cid
    base = tid * _CHUNK
    iota = lax.iota(jnp.int32, 16)

    pltpu.sync_copy(scores_hbm.at[pl.ds(base, _CHUNK)], sc_scores)

    def _mk_keys(i, carry):
        s = sc_scores[pl.ds(i * 16, 16)]
        b = lax.bitcast_convert_type(s, jnp.int32)
        pat = jnp.where(s >= 0.0, b | _MSB, ~b)
        sc_keys[pl.ds(i * 16, 16)] = pat
        return carry

    lax.fori_loop(0, _NV, _mk_keys, jnp.int32(0), unroll=8)

    need = jnp.int32(K_TOPK)
    prefix = jnp.int32(0)
    ones16 = jnp.ones((16,), jnp.int32)
    for r in range(8):
        shift = 28 - 4 * r
        plsc.subcore_barrier()
        for j in range(16):
            hist2d[j] = jnp.zeros((16,), jnp.int32)

        if r == 0:
            def _hist0(i, carry):
                k = sc_keys[pl.ds(i * 16, 16)]
                digit = lax.shift_right_logical(k, 28) & 0xF
                plsc.addupdate_scatter(hist2d, [digit, iota], ones16)
                return carry
            lax.fori_loop(0, _NV, _hist0, jnp.int32(0), unroll=8)
        else:
            def _hist(i, carry):
                pfx = carry
                k = sc_keys[pl.ds(i * 16, 16)]
                active = lax.shift_right_logical(k, shift + 4) == pfx
                digit = lax.shift_right_logical(k, shift) & 0xF
                plsc.addupdate_scatter(hist2d, [digit, iota], ones16,
                                       mask=active)
                return pfx
            lax.fori_loop(0, _NV, _hist, prefix, unroll=8)

        tl = jnp.zeros((16,), jnp.int32)
        for j in range(16):
            tl = tl + plsc.load_gather(hist2d, [iota, jnp.full((16,), j,
                                                               jnp.int32)])
        histbuf[...] = tl

        pltpu.sync_copy(histbuf, sh.at[pl.ds(_O_HIST + tid * 16, 16)])
        plsc.subcore_barrier()
        pltpu.sync_copy(sh.at[pl.ds(_O_HIST, 256)], fillbuf)
        total = jnp.zeros((16,), jnp.int32)
        for t in range(_NT):
            total = total + fillbuf[pl.ds(t * 16, 16)]
        cnt_ge = lax.rev(jnp.cumsum(lax.rev(total, (0,))), (0,))
        dsel = jnp.max(jnp.where(cnt_ge >= need, iota, -1))
        cnt_gt = jnp.sum(jnp.where(iota == dsel, cnt_ge - total, 0))
        need = need - cnt_gt
        prefix = (prefix << 4) | dsel

    T = prefix
    need_eq = need
    A = jnp.int32(K_TOPK) - need_eq

    Tx = T ^ _MSB

    def _select(i, carry):
        c_gt, c_eq = carry
        k = sc_keys[pl.ds(i * 16, 16)]
        kx = k ^ _MSB
        m_gt = kx > Tx
        m_eq = k == T
        gidx = base + i * 16 + iota
        mi_gt = m_gt.astype(jnp.int32)
        mi_eq = m_eq.astype(jnp.int32)
        pos_gt = c_gt + plsc.cumsum(mi_gt) - mi_gt
        pos_eq = c_eq + plsc.cumsum(mi_eq) - mi_eq
        m_gtw = m_gt & (pos_gt < 256)
        m_eqw = m_eq & (pos_eq < _EQ_CAP - 16)
        plsc.store_scatter(gt_idx, [pos_gt], gidx, mask=m_gtw)
        plsc.store_scatter(gt_key, [pos_gt], k, mask=m_gtw)
        plsc.store_scatter(eq_idx, [pos_eq], gidx, mask=m_eqw)
        c_gt = c_gt + jnp.sum(mi_gt)
        c_eq = c_eq + jnp.sum(mi_eq)
        return (c_gt, c_eq)

    c_gt, c_eq = lax.fori_loop(0, _NV, _select,
                               (jnp.int32(0), jnp.int32(0)), unroll=4)

    histbuf[...] = jnp.full((16,), c_gt, jnp.int32)
    pltpu.sync_copy(histbuf, sh.at[pl.ds(_O_CGT + tid * 16, 16)])
    histbuf[...] = jnp.full((16,), c_eq, jnp.int32)
    pltpu.sync_copy(histbuf, sh.at[pl.ds(_O_CEQ + tid * 16, 16)])
    pltpu.sync_copy(gt_key.at[pl.ds(0, 256)],
                    sh.at[pl.ds(_O_GTK + tid * 256, 256)])
    pltpu.sync_copy(gt_idx.at[pl.ds(0, 256)],
                    sh.at[pl.ds(_O_GTI + tid * 256, 256)])
    pltpu.sync_copy(eq_idx.at[pl.ds(0, 256)],
                    sh.at[pl.ds(_O_EQI + tid * 256, 256)])
    plsc.subcore_barrier()

    @pl.when(tid == 0)
    def _compact():
        pltpu.sync_copy(sh.at[pl.ds(_O_CGT, 256)], fillbuf)
        pltpu.sync_copy(sh.at[pl.ds(_O_CEQ, 256)], fillb2)
        pltpu.sync_copy(sh.at[pl.ds(_O_GTK, 4096)], gtk_all)
        pltpu.sync_copy(sh.at[pl.ds(_O_GTI, 4096)], gti_all)
        pltpu.sync_copy(sh.at[pl.ds(_O_EQI, 4096)], eqi_all)
        for i in range(16):
            cand_key[pl.ds(i * 16, 16)] = jnp.full((16,), T, jnp.int32)
        off = jnp.int32(0)
        for t in range(_NT):
            cg = jnp.max(fillbuf[pl.ds(t * 16, 16)])
            for j in range(16):
                p = j * 16 + iota
                pos = off + p
                m = (p < cg) & (pos >= 0) & (pos < 256)
                plsc.store_scatter(cand_key, [pos],
                                   gtk_all[pl.ds(t * 256 + j * 16, 16)],
                                   mask=m)
                plsc.store_scatter(cand_idx, [pos],
                                   gti_all[pl.ds(t * 256 + j * 16, 16)],
                                   mask=m)
            off = off + cg
        oeq = jnp.int32(0)
        for t in range(_NT):
            ce = jnp.max(fillb2[pl.ds(t * 16, 16)])
            for j in range(16):
                p = j * 16 + iota
                gpos = oeq + p
                pos = A + gpos
                m = ((p < ce) & (gpos < need_eq)
                     & (pos >= 0) & (pos < 256))
                plsc.store_scatter(cand_idx, [pos],
                                   eqi_all[pl.ds(t * 256 + j * 16, 16)],
                                   mask=m)
            oeq = oeq + ce
        for i in range(16):
            cand_key[pl.ds(256 + i * 16, 16)] = cand_key[pl.ds(i * 16, 16)]
            cand_idx[pl.ds(256 + i * 16, 16)] = cand_idx[pl.ds(i * 16, 16)]
        pltpu.sync_copy(cand_key, sh.at[pl.ds(_O_CNK, 512)])
        pltpu.sync_copy(cand_idx, sh.at[pl.ds(_O_CNI, 512)])

    plsc.subcore_barrier()
    pltpu.sync_copy(sh.at[pl.ds(_O_CNK, 512)], cand_key)
    pltpu.sync_copy(sh.at[pl.ds(_O_CNI, 512)], cand_idx)

    mybase = tid * 16
    key_me = cand_key[pl.ds(mybase, 16)]
    idx_me = cand_idx[pl.ds(mybase, 16)]
    kx_me = key_me ^ _MSB

    def _rank(s, rk):
        ks = cand_key[pl.ds(mybase + s, 16)]
        is_ = cand_idx[pl.ds(mybase + s, 16)]
        gt = (ks ^ _MSB) > kx_me
        tie = (ks == key_me) & (is_ < idx_me)
        return rk + (gt | tie).astype(jnp.int32)

    rank_me = lax.fori_loop(0, 256, _rank, jnp.zeros((16,), jnp.int32), unroll=8)

    smax = jnp.float32(-jnp.inf)
    for i in range(16):
        kk = cand_key[pl.ds(i * 16, 16)]
        b = jnp.where(kk < 0, kk ^ _MSB, ~kk)
        sv = lax.bitcast_convert_type(b, jnp.float32)
        smax = jnp.maximum(smax, jnp.max(sv))
    b_me = jnp.where(key_me < 0, key_me ^ _MSB, ~key_me)
    s_me = lax.bitcast_convert_type(b_me, jnp.float32)
    w_me = jnp.exp(s_me - smax)

    rank_ref[...] = rank_me
    widx_ref[...] = idx_me
    wbuf[...] = w_me
    pltpu.sync_copy(rank_ref, sh.at[pl.ds(_O_RNK + tid * 16, 16)])
    pltpu.sync_copy(widx_ref, sh.at[pl.ds(_O_RIX + tid * 16, 16)])
    histbuf[...] = lax.bitcast_convert_type(
        jnp.full((16,), jnp.sum(w_me), jnp.float32), jnp.int32)
    pltpu.sync_copy(histbuf, sh.at[pl.ds(_O_WSM + tid * 16, 16)])

    gidx_ref[...] = jnp.minimum(jnp.maximum(idx_me, 0), M_KV - 1)
    pltpu.async_copy(values_hbm.at[gidx_ref], vrows, dma_sem).wait()
    wv = wbuf[...]
    for c in range(8):
        acc[pl.ds(c * 16, 16)] = jnp.zeros((16,), jnp.float32)
    for l in range(16):
        wl = wv[l]
        for c in range(8):
            acc[pl.ds(c * 16, 16)] = (acc[pl.ds(c * 16, 16)]
                                      + wl * vrows[l, pl.ds(c * 16, 16)])
    for c in range(8):
        acc_i[pl.ds(c * 16, 16)] = lax.bitcast_convert_type(
            acc[pl.ds(c * 16, 16)], jnp.int32)
    pltpu.sync_copy(acc_i, sh.at[pl.ds(_O_ENC + tid * 128, 128)])
    plsc.subcore_barrier()

    @pl.when(tid == 0)
    def _finish():
        pltpu.sync_copy(sh.at[pl.ds(_O_WSM, 256)], fillbuf)
        denom = jnp.float32(0.0)
        for t in range(_NT):
            row = lax.bitcast_convert_type(fillbuf[pl.ds(t * 16, 16)],
                                           jnp.float32)
            denom = denom + jnp.max(row)
        pltpu.sync_copy(sh.at[pl.ds(_O_ENC, 2048)], gti_all.at[pl.ds(0, 2048)])
        for c in range(8):
            tot = jnp.zeros((16,), jnp.float32)
            for t in range(_NT):
                tot = tot + lax.bitcast_convert_type(
                    gti_all[pl.ds(t * 128 + c * 16, 16)], jnp.float32)
            acc[pl.ds(c * 16, 16)] = tot / denom
        pltpu.sync_copy(acc, out_enc)
        pltpu.sync_copy(sh.at[pl.ds(_O_RNK, 256)], fillbuf)
        pltpu.sync_copy(sh.at[pl.ds(_O_RIX, 256)], fillb2)
        for t in range(_NT):
            rv = fillbuf[pl.ds(t * 16, 16)]
            plsc.store_scatter(sc_keys, [rv], fillb2[pl.ds(t * 16, 16)],
                               mask=(rv >= 0) & (rv < 256))
        pltpu.sync_copy(sc_keys.at[pl.ds(0, 256)], out_idx)


def _sc_topk(scores, values):
    mesh = plsc.VectorSubcoreMesh(core_axis_name="c", subcore_axis_name="s",
                                  num_cores=1)
    f = pl.kernel(
        _sc_body,
        mesh=mesh,
        compiler_params=pltpu.CompilerParams(needs_layout_passes=False),
        out_type=[
            jax.ShapeDtypeStruct((DIM,), jnp.float32),
            jax.ShapeDtypeStruct((K_TOPK,), jnp.int32),
        ],
        scratch_types=[
            pltpu.VMEM((_CHUNK,), jnp.float32),
            pltpu.VMEM((_CHUNK,), jnp.int32),
            pltpu.VMEM((_GT_CAP,), jnp.int32),
            pltpu.VMEM((_GT_CAP,), jnp.int32),
            pltpu.VMEM((_EQ_CAP,), jnp.int32),
            pltpu.VMEM((512,), jnp.int32),
            pltpu.VMEM((512,), jnp.int32),
            pltpu.VMEM((16,), jnp.int32),
            pltpu.VMEM((16, 16), jnp.int32),
            pltpu.VMEM((16,), jnp.int32),
            pltpu.VMEM((16,), jnp.int32),
            pltpu.VMEM((16,), jnp.int32),
            pltpu.VMEM((16,), jnp.float32),
            pltpu.VMEM((16, DIM), jnp.float32),
            pltpu.VMEM((DIM,), jnp.float32),
            pltpu.VMEM((DIM,), jnp.int32),
            pltpu.VMEM((256,), jnp.int32),
            pltpu.VMEM((256,), jnp.int32),
            pltpu.VMEM((4096,), jnp.int32),
            pltpu.VMEM((4096,), jnp.int32),
            pltpu.VMEM((4096,), jnp.int32),
            pltpu.VMEM_SHARED((_SH_WORDS,), jnp.int32),
            pltpu.SemaphoreType.DMA,
        ],
    )
    return f(scores, values)


def kernel(queries, values, keys, affine):
    qp = jnp.matmul(queries, jnp.diag(affine))
    scores = _tc_scores(qp, keys)
    enc, idx = _sc_topk(scores, values)
    return enc, idx


# speedup vs baseline: 1.0074x; 1.0073x over previous
import jax
import jax.numpy as jnp
from jax import lax
from jax.experimental import pallas as pl
from jax.experimental.pallas import tpu as pltpu
from jax.experimental.pallas import tpu_sc as plsc

N_Q = 32
DIM = 128
M_KV = 32768
K_TOPK = 256


_BM = 2048


def _tc_scores_body(q_ref, k_ref, o_ref):
    qp = q_ref[...]
    kb = k_ref[...]
    s = lax.dot_general(qp, kb, (((1,), (1,)), ((), ())),
                        preferred_element_type=jnp.float32)
    o_ref[...] = jnp.max(s, axis=0)


def _tc_scores(qp, keys):
    return pl.pallas_call(
        _tc_scores_body,
        grid=(M_KV // _BM,),
        in_specs=[
            pl.BlockSpec((N_Q, DIM), lambda i: (0, 0)),
            pl.BlockSpec((_BM, DIM), lambda i: (i, 0)),
        ],
        out_specs=pl.BlockSpec((_BM,), lambda i: (i,)),
        out_shape=jax.ShapeDtypeStruct((M_KV,), jnp.float32),
    )(qp, keys)


_NT = 16
_CHUNK = M_KV // _NT
_NV = _CHUNK // 16
_MSB = -2**31
_GT_CAP = 288
_EQ_CAP = 2064

_O_HIST = 0
_O_CGT = 256
_O_CEQ = 512
_O_GTK = 768
_O_GTI = 4864
_O_EQI = 8960
_O_CNK = 13056
_O_CNI = 13568
_O_RNK = 14080
_O_RIX = 14336
_O_WSM = 14592
_O_ENC = 14848
_SH_WORDS = 16896


def _sc_body(scores_hbm, values_hbm, out_enc, out_idx,
             sc_scores, sc_keys, gt_idx, gt_key, eq_idx,
             cand_key, cand_idx, histbuf, hist256,
             rank_ref, widx_ref, gidx_ref, wbuf, vrows, acc, acc_i,
             fillbuf, fillb2, gtk_all, gti_all, eqi_all,
             sh, dma_sem):
    cid = lax.axis_index("c")
    sid = lax.axis_index("s")
    tid = sid * 1 + cid
    base = tid * _CHUNK
    iota = lax.iota(jnp.int32, 16)

    pltpu.sync_copy(scores_hbm.at[pl.ds(base, _CHUNK)], sc_scores)

    def _mk_keys(i, carry):
        s = sc_scores[pl.ds(i * 16, 16)]
        b = lax.bitcast_convert_type(s, jnp.int32)
        pat = jnp.where(s >= 0.0, b | _MSB, ~b)
        sc_keys[pl.ds(i * 16, 16)] = pat
        return carry

    lax.fori_loop(0, _NV, _mk_keys, jnp.int32(0), unroll=8)

    need = jnp.int32(K_TOPK)
    prefix = jnp.int32(0)
    ones16 = jnp.ones((16,), jnp.int32)
    for r in range(4):
        shift = 24 - 8 * r
        for j in range(16):
            hist256[pl.ds(j * 16, 16)] = jnp.zeros((16,), jnp.int32)

        if r == 0:
            def _hist0(i, carry):
                k = sc_keys[pl.ds(i * 16, 16)]
                digit = lax.shift_right_logical(k, 24) & 0xFF
                plsc.addupdate_scatter(hist256, [digit], ones16)
                return carry
            lax.fori_loop(0, _NV, _hist0, jnp.int32(0), unroll=8)
        else:
            def _hist(i, carry):
                pfx = carry
                k = sc_keys[pl.ds(i * 16, 16)]
                active = lax.shift_right_logical(k, shift + 8) == pfx
                digit = lax.shift_right_logical(k, shift) & 0xFF
                plsc.addupdate_scatter(hist256, [digit], ones16, mask=active)
                return pfx
            lax.fori_loop(0, _NV, _hist, prefix, unroll=8)

        pltpu.sync_copy(hist256, sh.at[pl.ds(_O_GTK + tid * 256, 256)])
        plsc.subcore_barrier()

        @pl.when(tid == 0)
        def _decide():
            pltpu.sync_copy(sh.at[pl.ds(_O_GTK, 4096)], gtk_all)
            tots = []
            for b in range(16):
                tv = jnp.zeros((16,), jnp.int32)
                for t in range(_NT):
                    tv = tv + gtk_all[pl.ds(t * 256 + b * 16, 16)]
                tots.append(tv)
            dsel = jnp.int32(-1)
            cntgt = jnp.int32(0)
            sfx_above = jnp.int32(0)
            for b in reversed(range(16)):
                sfx_in = lax.rev(jnp.cumsum(lax.rev(tots[b], (0,))), (0,))
                cge = sfx_in + sfx_above
                bmax = jnp.max(jnp.where(cge >= need, b * 16 + iota, -1))
                bcnt = jnp.sum(jnp.where(b * 16 + iota == bmax,
                                         cge - tots[b], 0))
                better = bmax > dsel
                dsel = jnp.where(better, bmax, dsel)
                cntgt = jnp.where(better, bcnt, cntgt)
                sfx_above = sfx_above + jnp.sum(tots[b])
            decv = (jnp.where(iota == 0, dsel, 0)
                    + jnp.where(iota == 1, cntgt, 0))
            histbuf[...] = decv
            pltpu.sync_copy(histbuf, sh.at[pl.ds(_O_HIST, 16)])

        plsc.subcore_barrier()
        pltpu.sync_copy(sh.at[pl.ds(_O_HIST, 16)], histbuf)
        dv = histbuf[...]
        dsel_all = jnp.sum(jnp.where(iota == 0, dv, 0))
        cntgt_all = jnp.sum(jnp.where(iota == 1, dv, 0))
        need = need - cntgt_all
        prefix = (prefix << 8) | dsel_all

    T = prefix
    need_eq = need
    A = jnp.int32(K_TOPK) - need_eq

    Tx = T ^ _MSB

    def _select(i, carry):
        c_gt, c_eq = carry
        k = sc_keys[pl.ds(i * 16, 16)]
        kx = k ^ _MSB
        m_gt = kx > Tx
        m_eq = k == T
        gidx = base + i * 16 + iota
        mi_gt = m_gt.astype(jnp.int32)
        mi_eq = m_eq.astype(jnp.int32)
        pos_gt = c_gt + plsc.cumsum(mi_gt) - mi_gt
        pos_eq = c_eq + plsc.cumsum(mi_eq) - mi_eq
        m_gtw = m_gt & (pos_gt < 256)
        m_eqw = m_eq & (pos_eq < _EQ_CAP - 16)
        plsc.store_scatter(gt_idx, [pos_gt], gidx, mask=m_gtw)
        plsc.store_scatter(gt_key, [pos_gt], k, mask=m_gtw)
        plsc.store_scatter(eq_idx, [pos_eq], gidx, mask=m_eqw)
        c_gt = c_gt + jnp.sum(mi_gt)
        c_eq = c_eq + jnp.sum(mi_eq)
        return (c_gt, c_eq)

    c_gt, c_eq = lax.fori_loop(0, _NV, _select,
                               (jnp.int32(0), jnp.int32(0)), unroll=4)

    histbuf[...] = jnp.full((16,), c_gt, jnp.int32)
    pltpu.sync_copy(histbuf, sh.at[pl.ds(_O_CGT + tid * 16, 16)])
    histbuf[...] = jnp.full((16,), c_eq, jnp.int32)
    pltpu.sync_copy(histbuf, sh.at[pl.ds(_O_CEQ + tid * 16, 16)])
    pltpu.sync_copy(gt_key.at[pl.ds(0, 256)],
                    sh.at[pl.ds(_O_GTK + tid * 256, 256)])
    pltpu.sync_copy(gt_idx.at[pl.ds(0, 256)],
                    sh.at[pl.ds(_O_GTI + tid * 256, 256)])
    pltpu.sync_copy(eq_idx.at[pl.ds(0, 256)],
                    sh.at[pl.ds(_O_EQI + tid * 256, 256)])
    plsc.subcore_barrier()

    @pl.when(tid == 0)
    def _compact():
        pltpu.sync_copy(sh.at[pl.ds(_O_CGT, 256)], fillbuf)
        pltpu.sync_copy(sh.at[pl.ds(_O_CEQ, 256)], fillb2)
        pltpu.sync_copy(sh.at[pl.ds(_O_GTK, 4096)], gtk_all)
        pltpu.sync_copy(sh.at[pl.ds(_O_GTI, 4096)], gti_all)
        pltpu.sync_copy(sh.at[pl.ds(_O_EQI, 4096)], eqi_all)
        for i in range(16):
            cand_key[pl.ds(i * 16, 16)] = jnp.full((16,), T, jnp.int32)
        off = jnp.int32(0)
        for t in range(_NT):
            cg = jnp.max(fillbuf[pl.ds(t * 16, 16)])
            for j in range(16):
                p = j * 16 + iota
                pos = off + p
                m = (p < cg) & (pos >= 0) & (pos < 256)
                plsc.store_scatter(cand_key, [pos],
                                   gtk_all[pl.ds(t * 256 + j * 16, 16)],
                                   mask=m)
                plsc.store_scatter(cand_idx, [pos],
                                   gti_all[pl.ds(t * 256 + j * 16, 16)],
                                   mask=m)
            off = off + cg
        oeq = jnp.int32(0)
        for t in range(_NT):
            ce = jnp.max(fillb2[pl.ds(t * 16, 16)])
            for j in range(16):
                p = j * 16 + iota
                gpos = oeq + p
                pos = A + gpos
                m = ((p < ce) & (gpos < need_eq)
                     & (pos >= 0) & (pos < 256))
                plsc.store_scatter(cand_idx, [pos],
                                   eqi_all[pl.ds(t * 256 + j * 16, 16)],
                                   mask=m)
            oeq = oeq + ce
        for i in range(16):
            cand_key[pl.ds(256 + i * 16, 16)] = cand_key[pl.ds(i * 16, 16)]
            cand_idx[pl.ds(256 + i * 16, 16)] = cand_idx[pl.ds(i * 16, 16)]
        pltpu.sync_copy(cand_key, sh.at[pl.ds(_O_CNK, 512)])
        pltpu.sync_copy(cand_idx, sh.at[pl.ds(_O_CNI, 512)])

    plsc.subcore_barrier()
    pltpu.sync_copy(sh.at[pl.ds(_O_CNK, 512)], cand_key)
    pltpu.sync_copy(sh.at[pl.ds(_O_CNI, 512)], cand_idx)

    mybase = tid * 16
    key_me = cand_key[pl.ds(mybase, 16)]
    idx_me = cand_idx[pl.ds(mybase, 16)]
    kx_me = key_me ^ _MSB

    def _rank(s, rk):
        ks = cand_key[pl.ds(mybase + s, 16)]
        is_ = cand_idx[pl.ds(mybase + s, 16)]
        gt = (ks ^ _MSB) > kx_me
        tie = (ks == key_me) & (is_ < idx_me)
        return rk + (gt | tie).astype(jnp.int32)

    rank_me = lax.fori_loop(0, 256, _rank, jnp.zeros((16,), jnp.int32), unroll=8)

    smax = jnp.float32(-jnp.inf)
    for i in range(16):
        kk = cand_key[pl.ds(i * 16, 16)]
        b = jnp.where(kk < 0, kk ^ _MSB, ~kk)
        sv = lax.bitcast_convert_type(b, jnp.float32)
        smax = jnp.maximum(smax, jnp.max(sv))
    b_me = jnp.where(key_me < 0, key_me ^ _MSB, ~key_me)
    s_me = lax.bitcast_convert_type(b_me, jnp.float32)
    w_me = jnp.exp(s_me - smax)

    rank_ref[...] = rank_me
    widx_ref[...] = idx_me
    wbuf[...] = w_me
    pltpu.sync_copy(rank_ref, sh.at[pl.ds(_O_RNK + tid * 16, 16)])
    pltpu.sync_copy(widx_ref, sh.at[pl.ds(_O_RIX + tid * 16, 16)])
    histbuf[...] = lax.bitcast_convert_type(
        jnp.full((16,), jnp.sum(w_me), jnp.float32), jnp.int32)
    pltpu.sync_copy(histbuf, sh.at[pl.ds(_O_WSM + tid * 16, 16)])

    gidx_ref[...] = jnp.minimum(jnp.maximum(idx_me, 0), M_KV - 1)
    pltpu.async_copy(values_hbm.at[gidx_ref], vrows, dma_sem).wait()
    wv = wbuf[...]
    for c in range(8):
        acc[pl.ds(c * 16, 16)] = jnp.zeros((16,), jnp.float32)
    for l in range(16):
        wl = wv[l]
        for c in range(8):
            acc[pl.ds(c * 16, 16)] = (acc[pl.ds(c * 16, 16)]
                                      + wl * vrows[l, pl.ds(c * 16, 16)])
    for c in range(8):
        acc_i[pl.ds(c * 16, 16)] = lax.bitcast_convert_type(
            acc[pl.ds(c * 16, 16)], jnp.int32)
    pltpu.sync_copy(acc_i, sh.at[pl.ds(_O_ENC + tid * 128, 128)])
    plsc.subcore_barrier()

    @pl.when(tid == 0)
    def _finish():
        pltpu.sync_copy(sh.at[pl.ds(_O_WSM, 256)], fillbuf)
        denom = jnp.float32(0.0)
        for t in range(_NT):
            row = lax.bitcast_convert_type(fillbuf[pl.ds(t * 16, 16)],
                                           jnp.float32)
            denom = denom + jnp.max(row)
        pltpu.sync_copy(sh.at[pl.ds(_O_ENC, 2048)], gti_all.at[pl.ds(0, 2048)])
        for c in range(8):
            tot = jnp.zeros((16,), jnp.float32)
            for t in range(_NT):
                tot = tot + lax.bitcast_convert_type(
                    gti_all[pl.ds(t * 128 + c * 16, 16)], jnp.float32)
            acc[pl.ds(c * 16, 16)] = tot / denom
        pltpu.sync_copy(acc, out_enc)
        pltpu.sync_copy(sh.at[pl.ds(_O_RNK, 256)], fillbuf)
        pltpu.sync_copy(sh.at[pl.ds(_O_RIX, 256)], fillb2)
        for t in range(_NT):
            rv = fillbuf[pl.ds(t * 16, 16)]
            plsc.store_scatter(sc_keys, [rv], fillb2[pl.ds(t * 16, 16)],
                               mask=(rv >= 0) & (rv < 256))
        pltpu.sync_copy(sc_keys.at[pl.ds(0, 256)], out_idx)


def _sc_topk(scores, values):
    mesh = plsc.VectorSubcoreMesh(core_axis_name="c", subcore_axis_name="s",
                                  num_cores=1)
    f = pl.kernel(
        _sc_body,
        mesh=mesh,
        compiler_params=pltpu.CompilerParams(needs_layout_passes=False),
        out_type=[
            jax.ShapeDtypeStruct((DIM,), jnp.float32),
            jax.ShapeDtypeStruct((K_TOPK,), jnp.int32),
        ],
        scratch_types=[
            pltpu.VMEM((_CHUNK,), jnp.float32),
            pltpu.VMEM((_CHUNK,), jnp.int32),
            pltpu.VMEM((_GT_CAP,), jnp.int32),
            pltpu.VMEM((_GT_CAP,), jnp.int32),
            pltpu.VMEM((_EQ_CAP,), jnp.int32),
            pltpu.VMEM((512,), jnp.int32),
            pltpu.VMEM((512,), jnp.int32),
            pltpu.VMEM((16,), jnp.int32),
            pltpu.VMEM((256,), jnp.int32),
            pltpu.VMEM((16,), jnp.int32),
            pltpu.VMEM((16,), jnp.int32),
            pltpu.VMEM((16,), jnp.int32),
            pltpu.VMEM((16,), jnp.float32),
            pltpu.VMEM((16, DIM), jnp.float32),
            pltpu.VMEM((DIM,), jnp.float32),
            pltpu.VMEM((DIM,), jnp.int32),
            pltpu.VMEM((256,), jnp.int32),
            pltpu.VMEM((256,), jnp.int32),
            pltpu.VMEM((4096,), jnp.int32),
            pltpu.VMEM((4096,), jnp.int32),
            pltpu.VMEM((4096,), jnp.int32),
            pltpu.VMEM_SHARED((_SH_WORDS,), jnp.int32),
            pltpu.SemaphoreType.DMA,
        ],
    )
    return f(scores, values)


def kernel(queries, values, keys, affine):
    qp = jnp.matmul(queries, jnp.diag(affine))
    scores = _tc_scores(qp, keys)
    enc, idx = _sc_topk(scores, values)
    return enc, idx
